# baseline (speedup 1.0000x reference)
import dataclasses
import functools

import jax
import jax.numpy as jnp
from jax import lax
from jax.experimental import pallas as pl
from jax.experimental.pallas import tpu as pltpu
from jax.experimental.pallas import tpu_sc as plsc

N = 50000
E = 800000
D = 32
D_REVIEW = 64

NC = 2
NS = 16
NW = NC * NS
L = 16

C = 1280
NCHUNKS = E // C
G = C // L
NPAIR = NCHUNKS // 2
NPAD = 50176
HLEN = 50048
HSTEP = 2944
OSTEP = 6272


def _sc_scatter(hT, rfwT, src, dst, w):
    mesh = plsc.VectorSubcoreMesh(core_axis_name="c", subcore_axis_name="s")
    cp = pltpu.CompilerParams()
    if "needs_layout_passes" in pltpu.CompilerParams.__dataclass_fields__:
        cp = dataclasses.replace(cp, needs_layout_passes=False)

    @functools.partial(
        pl.kernel,
        out_type=jax.ShapeDtypeStruct((D, 1, NPAD), jnp.float32),
        mesh=mesh,
        compiler_params=cp,
        scratch_types=[
            pltpu.VMEM((HLEN,), jnp.float32),
            pltpu.VMEM((NPAD,), jnp.float32),
            pltpu.VMEM((C,), jnp.int32),
            pltpu.VMEM((C,), jnp.int32),
            pltpu.VMEM((C,), jnp.float32),
            pltpu.VMEM((C,), jnp.float32),
            pltpu.VMEM((C,), jnp.int32),
            pltpu.VMEM((C,), jnp.int32),
            pltpu.VMEM((C,), jnp.float32),
            pltpu.VMEM((C,), jnp.float32),
            pltpu.SemaphoreType.DMA,
            pltpu.SemaphoreType.DMA,
        ],
    )
    def body(hT_hbm, rfwT_hbm, src_hbm, dst_hbm, w_hbm, out_hbm,
             hcol_v, acc_v, src_a, dst_a, w_a, r_a, src_b, dst_b, w_b, r_b,
             sem_a, sem_b):
        core = lax.axis_index("c")
        sub = lax.axis_index("s")
        f = sub * NC + core

        for q in range(HLEN // HSTEP):
            pltpu.sync_copy(hT_hbm.at[f, 0, pl.ds(q * HSTEP, HSTEP)],
                            hcol_v.at[pl.ds(q * HSTEP, HSTEP)])

        def zrow(i, carry):
            acc_v[pl.ds(i * L, L)] = jnp.zeros((L,), jnp.float32)
            return carry

        lax.fori_loop(0, NPAD // L, zrow, 0)

        bufs = {0: (src_a, dst_a, w_a, r_a, sem_a),
                1: (src_b, dst_b, w_b, r_b, sem_b)}

        def fire(c, which):
            sv, dv, wv, rv, sm = bufs[which]
            base = pl.multiple_of(c * C, 8)
            pltpu.async_copy(src_hbm.at[pl.ds(base, C)], sv, sm)
            pltpu.async_copy(dst_hbm.at[pl.ds(base, C)], dv, sm)
            pltpu.async_copy(w_hbm.at[pl.ds(base, C)], wv, sm)
            pltpu.async_copy(rfwT_hbm.at[f, 0, pl.ds(base, C)], rv, sm)

        def drain_compute(which):
            sv, dv, wv, rv, sm = bufs[which]
            pltpu.make_async_copy(src_hbm.at[pl.ds(0, C)], sv, sm).wait()
            pltpu.make_async_copy(dst_hbm.at[pl.ds(0, C)], dv, sm).wait()
            pltpu.make_async_copy(w_hbm.at[pl.ds(0, C)], wv, sm).wait()
            pltpu.make_async_copy(rfwT_hbm.at[f, 0, pl.ds(0, C)], rv, sm).wait()

            @pl.loop(0, G, unroll=16)
            def grp(g):
                sl = pl.ds(g * L, L)
                hv = plsc.load_gather(hcol_v, [sv[sl]])
                m = hv * wv[sl] + rv[sl]
                plsc.addupdate_scatter(acc_v, [dv[sl]], m)

        fire(0, 0)

        @pl.loop(0, NPAIR)
        def pair(i):
            fire(2 * i + 1, 1)
            drain_compute(0)

            @pl.when(i + 1 < NPAIR)
            def _():
                fire(2 * i + 2, 0)

            drain_compute(1)

        fire(NCHUNKS - 1, 0)
        drain_compute(0)

        for q in range(NPAD // OSTEP):
            pltpu.sync_copy(acc_v.at[pl.ds(q * OSTEP, OSTEP)],
                            out_hbm.at[f, 0, pl.ds(q * OSTEP, OSTEP)])

    return body(hT.reshape(D, 1, HLEN), rfwT.reshape(D, 1, E), src, dst, w)


def _hT_tc(x, node_W):

    def body(w_ref, x_ref, o_ref):
        o_ref[...] = lax.dot_general(
            w_ref[...], x_ref[...], (((1,), (1,)), ((), ())),
            preferred_element_type=jnp.float32)

    return pl.pallas_call(
        body,
        out_shape=jax.ShapeDtypeStruct((D, HLEN), jnp.float32),
    )(node_W, x)


def _rfwT_tc(review_feat, review_W, edge_w):
    BLK = 6400

    def body(w_ref, rf_ref, ew_ref, o_ref):
        prod = lax.dot_general(
            w_ref[...], rf_ref[...], (((1,), (1,)), ((), ())),
            preferred_element_type=jnp.float32)
        o_ref[...] = prod * ew_ref[...]

    return pl.pallas_call(
        body,
        grid=(E // BLK,),
        in_specs=[
            pl.BlockSpec((D, D_REVIEW), lambda i: (0, 0)),
            pl.BlockSpec((BLK, D_REVIEW), lambda i: (i, 0)),
            pl.BlockSpec((1, BLK), lambda i: (0, i)),
        ],
        out_specs=pl.BlockSpec((D, BLK), lambda i: (0, i)),
        out_shape=jax.ShapeDtypeStruct((D, E), jnp.float32),
    )(review_W, review_feat, edge_w.reshape(1, E))


def _tr_tc(a):

    def body(a_ref, o_ref):
        o_ref[...] = a_ref[...].T

    return pl.pallas_call(
        body,
        out_shape=jax.ShapeDtypeStruct((N, D), jnp.float32),
    )(a)


def kernel(x, edge_index, review_feat, edge_w, node_W, review_W):
    xp = jnp.zeros((HLEN, D), jnp.float32).at[:N, :].set(x)
    hT = _hT_tc(xp, node_W)
    rfwT = _rfwT_tc(review_feat, review_W, edge_w)
    src = edge_index[0].astype(jnp.int32)
    dst = edge_index[1].astype(jnp.int32)
    w = edge_w[:, 0]
    outT = _sc_scatter(hT, rfwT, src, dst, w)
    return _tr_tc(outT.reshape(D, NPAD)[:, :N])


# speedup vs baseline: 1.2037x; 1.2037x over previous
import dataclasses
import functools

import jax
import jax.numpy as jnp
from jax import lax
from jax.experimental import pallas as pl
from jax.experimental.pallas import tpu as pltpu
from jax.experimental.pallas import tpu_sc as plsc

N = 50000
E = 800000
D = 32
D_REVIEW = 64

NC = 2
NS = 16
NW = NC * NS
L = 16

C = 1280
NCHUNKS = E // C
G = C // L
NPAIR = NCHUNKS // 2
NPAD = 50176
HLEN = 50048
HSTEP = 2944
OSTEP = 6272


def _sc_scatter(hT, rfwT, src, dst, w):
    mesh = plsc.VectorSubcoreMesh(core_axis_name="c", subcore_axis_name="s")
    cp = pltpu.CompilerParams()
    if "needs_layout_passes" in pltpu.CompilerParams.__dataclass_fields__:
        cp = dataclasses.replace(cp, needs_layout_passes=False)

    @functools.partial(
        pl.kernel,
        out_type=jax.ShapeDtypeStruct((D, 1, NPAD), jnp.float32),
        mesh=mesh,
        compiler_params=cp,
        scratch_types=[
            pltpu.VMEM((HLEN,), jnp.float32),
            pltpu.VMEM((NPAD,), jnp.float32),
            pltpu.VMEM((C,), jnp.int32),
            pltpu.VMEM((C,), jnp.int32),
            pltpu.VMEM((C,), jnp.float32),
            pltpu.VMEM((C,), jnp.float32),
            pltpu.VMEM((C,), jnp.int32),
            pltpu.VMEM((C,), jnp.int32),
            pltpu.VMEM((C,), jnp.float32),
            pltpu.VMEM((C,), jnp.float32),
            pltpu.SemaphoreType.DMA,
            pltpu.SemaphoreType.DMA,
        ],
    )
    def body(hT_hbm, rfwT_hbm, src_hbm, dst_hbm, w_hbm, out_hbm,
             hcol_v, acc_v, src_a, dst_a, w_a, r_a, src_b, dst_b, w_b, r_b,
             sem_a, sem_b):
        core = lax.axis_index("c")
        sub = lax.axis_index("s")
        f = sub * NC + core

        for q in range(HLEN // HSTEP):
            pltpu.sync_copy(hT_hbm.at[f, 0, pl.ds(q * HSTEP, HSTEP)],
                            hcol_v.at[pl.ds(q * HSTEP, HSTEP)])

        def zrow(i, carry):
            acc_v[pl.ds(i * L, L)] = jnp.zeros((L,), jnp.float32)
            return carry

        lax.fori_loop(0, NPAD // L, zrow, 0)

        bufs = {0: (src_a, dst_a, w_a, r_a, sem_a),
                1: (src_b, dst_b, w_b, r_b, sem_b)}

        def fire(c, which):
            sv, dv, wv, rv, sm = bufs[which]
            base = pl.multiple_of(c * C, 8)
            pltpu.async_copy(src_hbm.at[pl.ds(base, C)], sv, sm)
            pltpu.async_copy(dst_hbm.at[pl.ds(base, C)], dv, sm)
            pltpu.async_copy(w_hbm.at[pl.ds(base, C)], wv, sm)
            pltpu.async_copy(rfwT_hbm.at[f, 0, pl.ds(base, C)], rv, sm)

        def drain_compute(which):
            sv, dv, wv, rv, sm = bufs[which]
            pltpu.make_async_copy(src_hbm.at[pl.ds(0, C)], sv, sm).wait()
            pltpu.make_async_copy(dst_hbm.at[pl.ds(0, C)], dv, sm).wait()
            pltpu.make_async_copy(w_hbm.at[pl.ds(0, C)], wv, sm).wait()
            pltpu.make_async_copy(rfwT_hbm.at[f, 0, pl.ds(0, C)], rv, sm).wait()

            @plsc.parallel_loop(0, G, unroll=8)
            def grp(g):
                sl = pl.ds(g * L, L)
                hv = plsc.load_gather(hcol_v, [sv[sl]])
                m = hv * wv[sl] + rv[sl]
                plsc.addupdate_scatter(acc_v, [dv[sl]], m)

        fire(0, 0)

        @pl.loop(0, NPAIR)
        def pair(i):
            fire(2 * i + 1, 1)
            drain_compute(0)

            @pl.when(i + 1 < NPAIR)
            def _():
                fire(2 * i + 2, 0)

            drain_compute(1)

        fire(NCHUNKS - 1, 0)
        drain_compute(0)

        for q in range(NPAD // OSTEP):
            pltpu.sync_copy(acc_v.at[pl.ds(q * OSTEP, OSTEP)],
                            out_hbm.at[f, 0, pl.ds(q * OSTEP, OSTEP)])

    return body(hT.reshape(D, 1, HLEN), rfwT.reshape(D, 1, E), src, dst, w)


def _hT_tc(x, node_W):

    def body(w_ref, x_ref, o_ref):
        o_ref[...] = lax.dot_general(
            w_ref[...], x_ref[...], (((1,), (1,)), ((), ())),
            preferred_element_type=jnp.float32)

    return pl.pallas_call(
        body,
        out_shape=jax.ShapeDtypeStruct((D, HLEN), jnp.float32),
    )(node_W, x)


def _rfwT_tc(review_feat, review_W, edge_w):
    BLK = 6400

    def body(w_ref, rf_ref, ew_ref, o_ref):
        prod = lax.dot_general(
            w_ref[...], rf_ref[...], (((1,), (1,)), ((), ())),
            preferred_element_type=jnp.float32)
        o_ref[...] = prod * ew_ref[...]

    return pl.pallas_call(
        body,
        grid=(E // BLK,),
        in_specs=[
            pl.BlockSpec((D, D_REVIEW), lambda i: (0, 0)),
            pl.BlockSpec((BLK, D_REVIEW), lambda i: (i, 0)),
            pl.BlockSpec((1, BLK), lambda i: (0, i)),
        ],
        out_specs=pl.BlockSpec((D, BLK), lambda i: (0, i)),
        out_shape=jax.ShapeDtypeStruct((D, E), jnp.float32),
    )(review_W, review_feat, edge_w.reshape(1, E))


def _tr_tc(a):

    def body(a_ref, o_ref):
        o_ref[...] = a_ref[...].T

    return pl.pallas_call(
        body,
        out_shape=jax.ShapeDtypeStruct((N, D), jnp.float32),
    )(a)


def kernel(x, edge_index, review_feat, edge_w, node_W, review_W):
    xp = jnp.zeros((HLEN, D), jnp.float32).at[:N, :].set(x)
    hT = _hT_tc(xp, node_W)
    rfwT = _rfwT_tc(review_feat, review_W, edge_w)
    src = edge_index[0].astype(jnp.int32)
    dst = edge_index[1].astype(jnp.int32)
    w = edge_w[:, 0]
    outT = _sc_scatter(hT, rfwT, src, dst, w)
    return _tr_tc(outT.reshape(D, NPAD)[:, :N])


# speedup vs baseline: 1.2044x; 1.0006x over previous
import dataclasses
import functools

import jax
import jax.numpy as jnp
from jax import lax
from jax.experimental import pallas as pl
from jax.experimental.pallas import tpu as pltpu
from jax.experimental.pallas import tpu_sc as plsc

N = 50000
E = 800000
D = 32
D_REVIEW = 64

NC = 2
NS = 16
NW = NC * NS
L = 16

C = 1280
NCHUNKS = E // C
G = C // L
NPAIR = NCHUNKS // 2
NPAD = 50176
HLEN = 50048
HSTEP = 2944
OSTEP = 6272


def _sc_scatter(hT, rfwT, src, dst, w):
    mesh = plsc.VectorSubcoreMesh(core_axis_name="c", subcore_axis_name="s")
    cp = pltpu.CompilerParams()
    if "needs_layout_passes" in pltpu.CompilerParams.__dataclass_fields__:
        cp = dataclasses.replace(cp, needs_layout_passes=False)

    @functools.partial(
        pl.kernel,
        out_type=jax.ShapeDtypeStruct((D, 1, NPAD), jnp.float32),
        mesh=mesh,
        compiler_params=cp,
        scratch_types=[
            pltpu.VMEM((HLEN,), jnp.float32),
            pltpu.VMEM((NPAD,), jnp.float32),
            pltpu.VMEM((C,), jnp.int32),
            pltpu.VMEM((C,), jnp.int32),
            pltpu.VMEM((C,), jnp.float32),
            pltpu.VMEM((C,), jnp.float32),
            pltpu.VMEM((C,), jnp.int32),
            pltpu.VMEM((C,), jnp.int32),
            pltpu.VMEM((C,), jnp.float32),
            pltpu.VMEM((C,), jnp.float32),
            pltpu.SemaphoreType.DMA,
            pltpu.SemaphoreType.DMA,
        ],
    )
    def body(hT_hbm, rfwT_hbm, src_hbm, dst_hbm, w_hbm, out_hbm,
             hcol_v, acc_v, src_a, dst_a, w_a, r_a, src_b, dst_b, w_b, r_b,
             sem_a, sem_b):
        core = lax.axis_index("c")
        sub = lax.axis_index("s")
        f = sub * NC + core

        for q in range(HLEN // HSTEP):
            pltpu.sync_copy(hT_hbm.at[f, 0, pl.ds(q * HSTEP, HSTEP)],
                            hcol_v.at[pl.ds(q * HSTEP, HSTEP)])

        def zrow(i, carry):
            acc_v[pl.ds(i * L, L)] = jnp.zeros((L,), jnp.float32)
            return carry

        lax.fori_loop(0, NPAD // L, zrow, 0)

        bufs = {0: (src_a, dst_a, w_a, r_a, sem_a),
                1: (src_b, dst_b, w_b, r_b, sem_b)}

        def fire(c, which):
            sv, dv, wv, rv, sm = bufs[which]
            base = pl.multiple_of(c * C, 8)
            pltpu.async_copy(src_hbm.at[pl.ds(base, C)], sv, sm)
            pltpu.async_copy(dst_hbm.at[pl.ds(base, C)], dv, sm)
            pltpu.async_copy(w_hbm.at[pl.ds(base, C)], wv, sm)
            pltpu.async_copy(rfwT_hbm.at[f, 0, pl.ds(base, C)], rv, sm)

        def drain_compute(which):
            sv, dv, wv, rv, sm = bufs[which]
            pltpu.make_async_copy(src_hbm.at[pl.ds(0, C)], sv, sm).wait()
            pltpu.make_async_copy(dst_hbm.at[pl.ds(0, C)], dv, sm).wait()
            pltpu.make_async_copy(w_hbm.at[pl.ds(0, C)], wv, sm).wait()
            pltpu.make_async_copy(rfwT_hbm.at[f, 0, pl.ds(0, C)], rv, sm).wait()

            @plsc.parallel_loop(0, G, unroll=16)
            def grp(g):
                sl = pl.ds(g * L, L)
                hv = plsc.load_gather(hcol_v, [sv[sl]])
                m = hv * wv[sl] + rv[sl]
                plsc.addupdate_scatter(acc_v, [dv[sl]], m)

        fire(0, 0)

        @pl.loop(0, NPAIR)
        def pair(i):
            fire(2 * i + 1, 1)
            drain_compute(0)

            @pl.when(i + 1 < NPAIR)
            def _():
                fire(2 * i + 2, 0)

            drain_compute(1)

        fire(NCHUNKS - 1, 0)
        drain_compute(0)

        for q in range(NPAD // OSTEP):
            pltpu.sync_copy(acc_v.at[pl.ds(q * OSTEP, OSTEP)],
                            out_hbm.at[f, 0, pl.ds(q * OSTEP, OSTEP)])

    return body(hT.reshape(D, 1, HLEN), rfwT.reshape(D, 1, E), src, dst, w)


def _hT_tc(x, node_W):

    def body(w_ref, x_ref, o_ref):
        o_ref[...] = lax.dot_general(
            w_ref[...], x_ref[...], (((1,), (1,)), ((), ())),
            preferred_element_type=jnp.float32)

    return pl.pallas_call(
        body,
        out_shape=jax.ShapeDtypeStruct((D, HLEN), jnp.float32),
    )(node_W, x)


def _rfwT_tc(review_feat, review_W, edge_w):
    BLK = 6400

    def body(w_ref, rf_ref, ew_ref, o_ref):
        prod = lax.dot_general(
            w_ref[...], rf_ref[...], (((1,), (1,)), ((), ())),
            preferred_element_type=jnp.float32)
        o_ref[...] = prod * ew_ref[...]

    return pl.pallas_call(
        body,
        grid=(E // BLK,),
        in_specs=[
            pl.BlockSpec((D, D_REVIEW), lambda i: (0, 0)),
            pl.BlockSpec((BLK, D_REVIEW), lambda i: (i, 0)),
            pl.BlockSpec((1, BLK), lambda i: (0, i)),
        ],
        out_specs=pl.BlockSpec((D, BLK), lambda i: (0, i)),
        out_shape=jax.ShapeDtypeStruct((D, E), jnp.float32),
    )(review_W, review_feat, edge_w.reshape(1, E))


def _tr_tc(a):

    def body(a_ref, o_ref):
        o_ref[...] = a_ref[...].T

    return pl.pallas_call(
        body,
        out_shape=jax.ShapeDtypeStruct((N, D), jnp.float32),
    )(a)


def kernel(x, edge_index, review_feat, edge_w, node_W, review_W):
    xp = jnp.zeros((HLEN, D), jnp.float32).at[:N, :].set(x)
    hT = _hT_tc(xp, node_W)
    rfwT = _rfwT_tc(review_feat, review_W, edge_w)
    src = edge_index[0].astype(jnp.int32)
    dst = edge_index[1].astype(jnp.int32)
    w = edge_w[:, 0]
    outT = _sc_scatter(hT, rfwT, src, dst, w)
    return _tr_tc(outT.reshape(D, NPAD)[:, :N])
